# baseline (device time: 24507 ns/iter reference)
import jax
import jax.numpy as jnp
from jax import lax
from jax.experimental import pallas as pl
from jax.experimental.pallas import tpu as pltpu

F32 = jnp.float32
BF16 = jnp.bfloat16


def kernel(x, router, W1, W2):
    t_per, d = x.shape
    e_per = W1.shape[0]
    assert router.shape == (d, e_per)

    def body(x_ref, r_ref, w1_ref, w2_ref, out_ref,
             xs_send, xs_recv, r_recv, wt_send, wt_recv,
             ps_send, ps_recv, w1b, w2b, send_sems, recv_sems):
        my_x = lax.axis_index("x")
        my_y = lax.axis_index("y")
        peer = (my_x, 1 - my_y)

        barrier = pltpu.get_barrier_semaphore()
        pl.semaphore_signal(barrier, inc=1, device_id=peer,
                            device_id_type=pl.DeviceIdType.MESH)
        pl.semaphore_wait(barrier, 1)

        def exchange(slot, src, dst):
            rdma = pltpu.make_async_remote_copy(
                src_ref=src, dst_ref=dst,
                send_sem=send_sems.at[slot], recv_sem=recv_sems.at[slot],
                device_id=peer, device_id_type=pl.DeviceIdType.MESH)
            rdma.start()
            return rdma

        xs_send[...] = x_ref[...].astype(BF16)
        r_rdma = exchange(0, r_ref, r_recv)
        x_rdma = exchange(1, xs_send, xs_recv)

        def gates(rhs):
            return lax.dot_general(
                x_ref[...], rhs, (((1,), (0,)), ((), ())),
                precision=lax.Precision.HIGHEST, preferred_element_type=F32)

        gl = gates(r_ref[...])

        w1b[...] = w1_ref[...].astype(BF16)
        w2b[...] = w2_ref[...].astype(BF16)

        r_rdma.wait_recv()
        gr = gates(r_recv[...])

        a1 = jnp.max(gl, axis=1, keepdims=True)
        a2 = jnp.min(gl, axis=1, keepdims=True)
        b1 = jnp.max(gr, axis=1, keepdims=True)
        b2 = jnp.min(gr, axis=1, keepdims=True)
        m1 = jnp.maximum(a1, b1)
        m2 = jnp.maximum(jnp.minimum(a1, b1), jnp.where(a1 > b1, a2, b2))
        denom = 1.0 + jnp.exp(m2 - m1)
        w_loc = jnp.where(gl >= m2, jnp.exp(gl - m1), 0.0) / denom
        wt_send[...] = jnp.where(gr >= m2, jnp.exp(gr - m1), 0.0) / denom
        w_rdma = exchange(2, wt_send, wt_recv)

        def run_experts(xb, wts):
            acc = jnp.zeros((xb.shape[0], d), F32)
            for e in range(e_per):
                h = lax.dot_general(xb, w1b[e],
                                    (((1,), (0,)), ((), ())),
                                    preferred_element_type=F32)
                h = jnp.maximum(h, 0.0).astype(BF16)
                o = lax.dot_general(h, w2b[e],
                                    (((1,), (0,)), ((), ())),
                                    preferred_element_type=F32)
                acc = acc + o * wts[:, e:e + 1]
            return acc

        acc_mine = run_experts(xs_send[...], w_loc)

        x_rdma.wait_recv()
        w_rdma.wait_recv()
        ps_send[...] = run_experts(xs_recv[...], wt_recv[...]).astype(BF16)
        p_rdma = exchange(3, ps_send, ps_recv)

        p_rdma.wait_recv()
        out_ref[...] = acc_mine + ps_recv[...].astype(F32)

        for rdma in (r_rdma, x_rdma, w_rdma, p_rdma):
            rdma.wait_send()

    return pl.pallas_call(
        body,
        out_shape=jax.ShapeDtypeStruct((t_per, d), F32),
        in_specs=[pl.BlockSpec(memory_space=pltpu.VMEM)] * 4,
        out_specs=pl.BlockSpec(memory_space=pltpu.VMEM),
        scratch_shapes=[
            pltpu.VMEM((t_per, d), BF16),
            pltpu.VMEM((t_per, d), BF16),
            pltpu.VMEM((d, e_per), F32),
            pltpu.VMEM((t_per, e_per), F32),
            pltpu.VMEM((t_per, e_per), F32),
            pltpu.VMEM((t_per, d), BF16),
            pltpu.VMEM((t_per, d), BF16),
            pltpu.VMEM(W1.shape, BF16),
            pltpu.VMEM(W2.shape, BF16),
            pltpu.SemaphoreType.DMA((4,)),
            pltpu.SemaphoreType.DMA((4,)),
        ],
        compiler_params=pltpu.CompilerParams(collective_id=0),
    )(x, router, W1, W2)


# device time: 20247 ns/iter; 1.2104x vs baseline; 1.2104x over previous
import os

import jax
import jax.numpy as jnp
from jax import lax
from jax.experimental import pallas as pl
from jax.experimental.pallas import tpu as pltpu

F32 = jnp.float32
BF16 = jnp.bfloat16

C = int(os.environ.get("CHUNKS", "4"))
_TINY = os.environ.get("ABLATE") == "tiny"


def kernel(x, router, W1, W2):
    t_per, d = x.shape
    e_per = W1.shape[0]
    t_c = t_per // C
    assert router.shape == (d, e_per) and t_per % C == 0

    W1 = W1.astype(BF16)
    W2 = W2.astype(BF16)

    def body(x_ref, r_ref, w1_hbm, w2_hbm, out_ref,
             w1v, w2v, xs_send, xs_recv, r_recv, wt_send, wt_recv,
             ps_send, ps_recv, o_buf, copy_sems, r_sems, w_sems,
             xs_send_sems, xs_recv_sems, ps_send_sems, ps_recv_sems):
        my_x = lax.axis_index("x")
        my_y = lax.axis_index("y")
        peer = (my_x, 1 - my_y)

        w1_copy = pltpu.make_async_copy(w1_hbm, w1v, copy_sems.at[0])
        w2_copy = pltpu.make_async_copy(w2_hbm, w2v, copy_sems.at[1])
        w1_copy.start()
        w2_copy.start()

        with jax.named_scope("entry_barrier"):
            barrier = pltpu.get_barrier_semaphore()
            pl.semaphore_signal(barrier, inc=1, device_id=peer,
                                device_id_type=pl.DeviceIdType.MESH)
            pl.semaphore_wait(barrier, 1)

        def exchange(src, dst, send_sem, recv_sem):
            rdma = pltpu.make_async_remote_copy(
                src_ref=src, dst_ref=dst, send_sem=send_sem,
                recv_sem=recv_sem, device_id=peer,
                device_id_type=pl.DeviceIdType.MESH)
            rdma.start()
            return rdma

        with jax.named_scope("dispatch_send"):
            r_rdma = exchange(r_ref, r_recv, r_sems.at[0], r_sems.at[1])
            xs_send[...] = x_ref[...].astype(BF16)
            t_x = 8 if _TINY else t_c
            x_rdmas = [
                exchange(xs_send.at[pl.ds(c * t_c, t_x)],
                         xs_recv.at[pl.ds(c * t_c, t_x)],
                         xs_send_sems.at[c], xs_recv_sems.at[c])
                for c in range(C)
            ]

        def gates(rhs):
            return lax.dot_general(
                x_ref[...], rhs, (((1,), (0,)), ((), ())),
                precision=lax.Precision.HIGHEST, preferred_element_type=F32)

        with jax.named_scope("gates"):
            gl = gates(r_ref[...])
            r_rdma.wait_recv()
            gr = gates(r_recv[...])

        a1 = jnp.max(gl, axis=1, keepdims=True)
        a2 = jnp.min(gl, axis=1, keepdims=True)
        b1 = jnp.max(gr, axis=1, keepdims=True)
        b2 = jnp.min(gr, axis=1, keepdims=True)
        m1 = jnp.maximum(a1, b1)
        m2 = jnp.maximum(jnp.minimum(a1, b1), jnp.where(a1 > b1, a2, b2))
        denom = 1.0 + jnp.exp(m2 - m1)
        w_loc = jnp.where(gl >= m2, jnp.exp(gl - m1), 0.0) / denom
        wt_send[...] = jnp.where(gr >= m2, jnp.exp(gr - m1), 0.0) / denom
        w_rdma = exchange(wt_send, wt_recv, w_sems.at[0], w_sems.at[1])

        with jax.named_scope("wait_wcopy"):
            w1_copy.wait()
            w2_copy.wait()

        def expert_out(xb, e):
            h = lax.dot_general(xb, w1v[e], (((1,), (0,)), ((), ())),
                                preferred_element_type=F32)
            h = jnp.maximum(h, 0.0).astype(BF16)
            return lax.dot_general(h, w2v[e], (((1,), (0,)), ((), ())),
                                   preferred_element_type=F32)

        def run_experts(xb, wts):
            acc = jnp.zeros((xb.shape[0], d), F32)
            for e in range(e_per):
                acc = acc + expert_out(xb, e) * wts[:, e:e + 1]
            return acc

        for c in range(C):
            sl = pl.ds(c * t_c, t_c)
            with jax.named_scope(f"o_chunk#c={c}"):
                x_rdmas[c].wait_recv()
                for e in range(e_per):
                    o_buf[e, sl] = expert_out(xs_recv[sl], e)

        with jax.named_scope("wait_wrecv"):
            w_rdma.wait_recv()
        ps_rdmas = []
        for c in range(C):
            sl = pl.ds(c * t_c, t_c)
            with jax.named_scope(f"ps_chunk#c={c}"):
                ps = o_buf[0, sl] * wt_recv[sl][:, 0:1]
                for e in range(1, e_per):
                    ps = ps + o_buf[e, sl] * wt_recv[sl][:, e:e + 1]
                ps_send[sl] = ps.astype(BF16)
                ps_rdmas.append(
                    exchange(ps_send.at[pl.ds(c * t_c, t_x)],
                             ps_recv.at[pl.ds(c * t_c, t_x)],
                             ps_send_sems.at[c], ps_recv_sems.at[c]))

        with jax.named_scope("acc_mine"):
            out_ref[...] = run_experts(xs_send[...], w_loc)

        with jax.named_scope("combine_wait_add"):
            for c in range(C):
                sl = pl.ds(c * t_c, t_c)
                ps_rdmas[c].wait_recv()
                out_ref[sl, :] = out_ref[sl, :] + ps_recv[sl].astype(F32)

        for rdma in [r_rdma, w_rdma] + x_rdmas + ps_rdmas:
            rdma.wait_send()

    return pl.pallas_call(
        body,
        out_shape=jax.ShapeDtypeStruct((t_per, d), F32),
        in_specs=[
            pl.BlockSpec(memory_space=pltpu.VMEM),
            pl.BlockSpec(memory_space=pltpu.VMEM),
            pl.BlockSpec(memory_space=pl.ANY),
            pl.BlockSpec(memory_space=pl.ANY),
        ],
        out_specs=pl.BlockSpec(memory_space=pltpu.VMEM),
        scratch_shapes=[
            pltpu.VMEM(W1.shape, BF16),
            pltpu.VMEM(W2.shape, BF16),
            pltpu.VMEM((t_per, d), BF16),
            pltpu.VMEM((t_per, d), BF16),
            pltpu.VMEM((d, e_per), F32),
            pltpu.VMEM((t_per, e_per), F32),
            pltpu.VMEM((t_per, e_per), F32),
            pltpu.VMEM((t_per, d), BF16),
            pltpu.VMEM((t_per, d), BF16),
            pltpu.VMEM((e_per, t_per, d), F32),
            pltpu.SemaphoreType.DMA((2,)),
            pltpu.SemaphoreType.DMA((2,)),
            pltpu.SemaphoreType.DMA((2,)),
            pltpu.SemaphoreType.DMA((C,)),
            pltpu.SemaphoreType.DMA((C,)),
            pltpu.SemaphoreType.DMA((C,)),
            pltpu.SemaphoreType.DMA((C,)),
        ],
        compiler_params=pltpu.CompilerParams(collective_id=0),
    )(x, router, W1, W2)
